# baseline (device time: 21519 ns/iter reference)
import jax
import jax.numpy as jnp
from jax import lax
from jax.experimental import pallas as pl
from jax.experimental.pallas import tpu as pltpu

N_DEV = 4
BLOCK = 64
BF16 = jnp.bfloat16


def kernel(x, Wq, K_ext, V_ext, Wo):
    B, sq_loc, d_model = x.shape
    d_in, hd_loc = Wq.shape
    _, skv, hq, dh = K_ext.shape
    hd_tot = hq * dh
    d_out = Wo.shape[1]
    hq_loc = hd_loc // dh
    d_out_h = d_out // 2
    pk = d_in + d_out

    K2 = K_ext.reshape(B, skv, hd_tot)
    V2 = V_ext.reshape(B, skv, hd_tot)

    def body(x_ref, wq_ref, k_ref, v_ref, wo_ref, out_ref,
             comm, xb, kt, vt, ctx_blk,
             sr, rr, sl, rl):
        my = lax.axis_index("i")
        left = (my - 1) % N_DEV
        right = (my + 1) % N_DEV
        opp = (my + 2) % N_DEV

        barrier_sem = pltpu.get_barrier_semaphore()
        for nbr in (left, right):
            pl.semaphore_signal(
                barrier_sem, inc=1,
                device_id=(nbr,), device_id_type=pl.DeviceIdType.MESH,
            )
        pl.semaphore_wait(barrier_sem, 2)

        comm[my, 0:d_in, :] = wq_ref[...].astype(BF16)
        comm[my, d_in:d_in + hd_loc, :] = wo_ref[:, 0:d_out_h].astype(BF16)
        comm[my, d_in + hd_loc:pk, :] = wo_ref[:, d_out_h:d_out].astype(BF16)

        QH = slice(0, d_in)
        OH = slice(d_in, pk)

        def rdma(origin, rows, sem_i, send_sems, recv_sems, dst):
            sub = lambda ref: ref.at[origin, rows, :]
            return pltpu.make_async_remote_copy(
                src_ref=sub(comm), dst_ref=sub(comm),
                send_sem=send_sems.at[sem_i], recv_sem=recv_sems.at[sem_i],
                device_id=(dst,), device_id_type=pl.DeviceIdType.MESH,
            )

        q_rows = my * sq_loc + lax.broadcasted_iota(jnp.int32, (sq_loc, skv), 0)
        qb = q_rows // BLOCK
        kb = lax.broadcasted_iota(jnp.int32, (sq_loc, skv), 1) // BLOCK
        mask = (qb == kb) | (kb == 0) | ((qb + kb) % 3 == 0)
        bias = jnp.where(mask, 0.0, -1e9).astype(jnp.float32)

        def attn_chunk(origin):
            for b in range(B):
                q_all = jnp.dot(xb[b], comm[origin, QH, :],
                                preferred_element_type=jnp.float32
                                ).astype(BF16)
                kc = kt[b, origin]
                vc = vt[b, origin]
                for i in range(hq_loc):
                    q = q_all[:, i * dh:(i + 1) * dh]
                    k = kc[:, i * dh:(i + 1) * dh]
                    v = vc[:, i * dh:(i + 1) * dh]
                    s = lax.dot_general(
                        q, k, (((1,), (1,)), ((), ())),
                        preferred_element_type=jnp.float32,
                    )
                    w = jnp.exp(s + bias)
                    denom = jnp.sum(w, axis=-1, keepdims=True)
                    ctx = jnp.dot(w.astype(BF16), v,
                                  preferred_element_type=jnp.float32)
                    ctx_blk[b, :, i * dh:(i + 1) * dh] = (
                        ctx / denom).astype(BF16)

        def out_partial(origin, init):
            lc = comm[origin, d_in:d_in + hd_loc, :]
            rc = comm[origin, d_in + hd_loc:pk, :]
            for b in range(B):
                pl_ = jnp.dot(ctx_blk[b], lc,
                              preferred_element_type=jnp.float32)
                pr_ = jnp.dot(ctx_blk[b], rc,
                              preferred_element_type=jnp.float32)
                if init:
                    out_ref[b, :, 0:d_out_h] = pl_
                    out_ref[b, :, d_out_h:d_out] = pr_
                else:
                    out_ref[b, :, 0:d_out_h] = out_ref[b, :, 0:d_out_h] + pl_
                    out_ref[b, :, d_out_h:d_out] = (
                        out_ref[b, :, d_out_h:d_out] + pr_)

        h1 = [
            rdma(my, QH, 0, sr, rr, right),
            rdma(my, OH, 0, sl, rl, left),
            rdma(my, OH, 1, sr, rr, right),
            rdma(my, QH, 1, sl, rl, left),
        ]
        for d in h1:
            d.start()

        for b in range(B):
            xb[b] = (x_ref[b] * 0.125).astype(BF16)
            for c in range(N_DEV):
                kt[b, c] = k_ref[b, :, c * hd_loc:(c + 1) * hd_loc].astype(BF16)
                vt[b, c] = v_ref[b, :, c * hd_loc:(c + 1) * hd_loc].astype(BF16)

        rdma(left, QH, 0, sr, rr, right).wait_recv()
        fwd_r = rdma(left, QH, 2, sr, rr, right)
        fwd_r.start()
        rdma(right, OH, 0, sl, rl, left).wait_recv()
        fwd_l = rdma(right, OH, 2, sl, rl, left)
        fwd_l.start()

        attn_chunk(my)
        out_partial(my, init=True)

        attn_chunk(left)
        rdma(left, OH, 1, sr, rr, right).wait_recv()
        out_partial(left, init=False)

        rdma(right, QH, 1, sl, rl, left).wait_recv()
        attn_chunk(right)
        out_partial(right, init=False)

        rdma(opp, QH, 2, sr, rr, right).wait_recv()
        attn_chunk(opp)
        rdma(opp, OH, 2, sl, rl, left).wait_recv()
        out_partial(opp, init=False)

        for d in h1 + [fwd_r, fwd_l]:
            d.wait_send()

    return pl.pallas_call(
        body,
        out_shape=jax.ShapeDtypeStruct((B, sq_loc, d_out), jnp.float32),
        in_specs=[pl.BlockSpec(memory_space=pltpu.VMEM)] * 5,
        out_specs=pl.BlockSpec(memory_space=pltpu.VMEM),
        scratch_shapes=[
            pltpu.VMEM((N_DEV, pk, hd_loc), BF16),
            pltpu.VMEM((B, sq_loc, d_model), BF16),
            pltpu.VMEM((B, N_DEV, skv, hd_loc), BF16),
            pltpu.VMEM((B, N_DEV, skv, hd_loc), BF16),
            pltpu.VMEM((B, sq_loc, hd_loc), BF16),
            pltpu.SemaphoreType.DMA((3,)),
            pltpu.SemaphoreType.DMA((3,)),
            pltpu.SemaphoreType.DMA((3,)),
            pltpu.SemaphoreType.DMA((3,)),
        ],
        compiler_params=pltpu.CompilerParams(collective_id=0),
    )(x, Wq, K2, V2, Wo)
